# baseline (speedup 1.0000x reference)
import functools

import jax
import jax.numpy as jnp
from jax import lax
from jax.experimental import pallas as pl
from jax.experimental.pallas import tpu as pltpu
from jax.experimental.pallas import tpu_sc as plsc

HIDDEN = 64
SEQ = 20
NUM_WORKERS = 32
ROWS_PER_WIN = 32
TOK_PER_WIN = ROWS_PER_WIN * SEQ
CHUNK = 128
CHUNKS_PER_WIN = TOK_PER_WIN // CHUNK
TC_BLOCK = 128


def kernel(input_ids, W):
    B, S = input_ids.shape
    n = B * S
    n_wins = B // (NUM_WORKERS * ROWS_PER_WIN)
    idx = input_ids.reshape(n).astype(jnp.int32)
    table2 = W.reshape(W.shape[0] // 2, 2 * HIDDEN)

    mesh = plsc.VectorSubcoreMesh(core_axis_name="core",
                                  subcore_axis_name="subcore")

    @functools.partial(
        pl.kernel,
        out_type=jax.ShapeDtypeStruct((n * HIDDEN,), jnp.float32),
        mesh=mesh,
        scratch_types=[
            pltpu.VMEM((TOK_PER_WIN,), jnp.int32),
            pltpu.VMEM((TOK_PER_WIN,), jnp.int32),
            pltpu.VMEM((TOK_PER_WIN,), jnp.int32),
            pltpu.VMEM((TOK_PER_WIN,), jnp.int32),
            pltpu.VMEM((TOK_PER_WIN,), jnp.int32),
            pltpu.VMEM((TOK_PER_WIN,), jnp.int32),
            pltpu.VMEM((4, CHUNK, 2 * HIDDEN), jnp.float32),
            pltpu.VMEM((TOK_PER_WIN * HIDDEN,), jnp.float32),
            pltpu.SemaphoreType.DMA,
            pltpu.SemaphoreType.DMA,
            pltpu.SemaphoreType.DMA,
        ],
    )
    def gather_kernel(table_hbm, idx_hbm, out_hbm,
                      ibuf0, ibuf1, pbuf0, pbuf1, hbuf0, hbuf1,
                      pair, obuf, isem, gsem, osem):
        ibufs, pbufs, hbufs = (ibuf0, ibuf1), (pbuf0, pbuf1), (hbuf0, hbuf1)
        wid = lax.axis_index("subcore") * 2 + lax.axis_index("core")
        base_tok = wid * (n_wins * TOK_PER_WIN)

        def idx_copy(win, slot):
            return pltpu.make_async_copy(
                idx_hbm.at[pl.ds(base_tok + win * TOK_PER_WIN, TOK_PER_WIN)],
                ibufs[slot], isem)

        def gather_copy(c, islot, pslot):
            return pltpu.make_async_copy(
                table_hbm.at[pbufs[islot].at[pl.ds(c * CHUNK, CHUNK)]],
                pair.at[pslot], gsem)

        def out_copy(win):
            return pltpu.make_async_copy(
                obuf,
                out_hbm.at[pl.ds(pl.multiple_of(
                    (base_tok + win * TOK_PER_WIN) * HIDDEN,
                    TOK_PER_WIN * HIDDEN), TOK_PER_WIN * HIDDEN)],
                osem)

        def repack(slot):
            @pl.loop(0, TOK_PER_WIN, step=16)
            def _(g):
                v = ibufs[slot][pl.ds(g, 16)]
                pbufs[slot][pl.ds(g, 16)] = lax.shift_right_logical(v, 1)
                hbufs[slot][pl.ds(g, 16)] = (v & 1) * HIDDEN

        def extract(c, slot, pslot):
            @pl.loop(0, CHUNK, step=16)
            def _(g):
                hv = hbufs[slot][pl.ds(c * CHUNK + g, 16)]
                tok0 = (c * CHUNK + g) * HIDDEN
                for j in range(16):
                    h = hv[j]
                    for k in range(HIDDEN // 16):
                        obuf[pl.ds(tok0 + j * HIDDEN + 16 * k, 16)] = (
                            pair[pslot, g + j, pl.ds(h + 16 * k, 16)])

        def window(win, slot):
            idx_copy(win, slot).wait()

            @pl.when(win + 1 < n_wins)
            def _():
                idx_copy(win + 1, 1 - slot).start()

            repack(slot)
            for c in range(min(3, CHUNKS_PER_WIN)):
                gather_copy(c, slot, c % 4).start()

            @pl.when(win >= 1)
            def _():
                out_copy(win - 1).wait()

            for c in range(CHUNKS_PER_WIN):
                if c + 3 < CHUNKS_PER_WIN:
                    gather_copy(c + 3, slot, (c + 3) % 4).start()
                gather_copy(c, slot, c % 4).wait()
                extract(c, slot, c % 4)

            out_copy(win).start()

        idx_copy(0, 0).start()

        @pl.loop(0, n_wins, step=2)
        def _(win):
            window(win, 0)
            window(win + 1, 1)

        out_copy(n_wins - 1).wait()

    packed = gather_kernel(table2, idx)
    packed2 = packed.reshape(n // 2, 2 * HIDDEN)

    def reshape_body(x_ref, o_ref):
        x = x_ref[...]
        mid = x.reshape(TC_BLOCK, SEQ // 2, 2 * HIDDEN)
        lo = mid[:, :, :HIDDEN]
        hi = mid[:, :, HIDDEN:]
        st = jnp.stack([lo, hi], axis=2)
        o_ref[...] = st.reshape(TC_BLOCK, SEQ, HIDDEN)

    out = pl.pallas_call(
        reshape_body,
        grid=(B // TC_BLOCK,),
        in_specs=[pl.BlockSpec((TC_BLOCK * SEQ // 2, 2 * HIDDEN),
                               lambda i: (i, 0))],
        out_specs=pl.BlockSpec((TC_BLOCK, SEQ, HIDDEN), lambda i: (i, 0, 0)),
        out_shape=jax.ShapeDtypeStruct((B, S, HIDDEN), jnp.float32),
    )(packed2)
    return out


# speedup vs baseline: 1.1895x; 1.1895x over previous
import functools

import jax
import jax.numpy as jnp
from jax import lax
from jax.experimental import pallas as pl
from jax.experimental.pallas import tpu as pltpu
from jax.experimental.pallas import tpu_sc as plsc

HIDDEN = 64
SEQ = 20
NUM_WORKERS = 32
ROWS_PER_WIN = 32
TOK_PER_WIN = ROWS_PER_WIN * SEQ
CHUNK = 128
CHUNKS_PER_WIN = TOK_PER_WIN // CHUNK


def kernel(input_ids, W):
    B, S = input_ids.shape
    n = B * S
    n_wins = B // (NUM_WORKERS * ROWS_PER_WIN)
    idx = input_ids.reshape(n).astype(jnp.int32)
    table2 = W.reshape(W.shape[0] // 2, 2 * HIDDEN)

    mesh = plsc.VectorSubcoreMesh(core_axis_name="core",
                                  subcore_axis_name="subcore")

    @functools.partial(
        pl.kernel,
        out_type=jax.ShapeDtypeStruct((B, S, HIDDEN), jnp.float32),
        mesh=mesh,
        scratch_types=[
            pltpu.VMEM((TOK_PER_WIN,), jnp.int32),
            pltpu.VMEM((TOK_PER_WIN,), jnp.int32),
            pltpu.VMEM((TOK_PER_WIN,), jnp.int32),
            pltpu.VMEM((TOK_PER_WIN,), jnp.int32),
            pltpu.VMEM((TOK_PER_WIN,), jnp.int32),
            pltpu.VMEM((TOK_PER_WIN,), jnp.int32),
            pltpu.VMEM((2, CHUNK, 2 * HIDDEN), jnp.float32),
            pltpu.VMEM((TOK_PER_WIN, HIDDEN), jnp.float32),
            pltpu.SemaphoreType.DMA,
            pltpu.SemaphoreType.DMA,
            pltpu.SemaphoreType.DMA,
        ],
    )
    def gather_kernel(table_hbm, idx_hbm, out_hbm,
                      ibuf0, ibuf1, pbuf0, pbuf1, hbuf0, hbuf1,
                      pair, obuf, isem, gsem, osem):
        ibufs, pbufs, hbufs = (ibuf0, ibuf1), (pbuf0, pbuf1), (hbuf0, hbuf1)
        wid = lax.axis_index("subcore") * 2 + lax.axis_index("core")
        base_row = wid * (n_wins * ROWS_PER_WIN)
        base_tok = base_row * SEQ

        def idx_copy(win, slot):
            return pltpu.make_async_copy(
                idx_hbm.at[pl.ds(base_tok + win * TOK_PER_WIN, TOK_PER_WIN)],
                ibufs[slot], isem)

        def gather_start(c, islot, pslot):
            for q in range(CHUNK // 16):
                pv = pbufs[islot][pl.ds(c * CHUNK + 16 * q, 16)]
                pltpu.make_async_copy(
                    table_hbm.at[pv],
                    pair.at[pslot, pl.ds(16 * q, 16)], gsem).start()

        def gather_wait(pslot):
            pltpu.make_async_copy(
                table_hbm.at[pbufs[0].at[pl.ds(0, CHUNK)]],
                pair.at[pslot], gsem).wait()

        def out_copy(win):
            return pltpu.make_async_copy(
                obuf.reshape(ROWS_PER_WIN, SEQ, HIDDEN),
                out_hbm.at[pl.ds(base_row + win * ROWS_PER_WIN, ROWS_PER_WIN)],
                osem)

        def repack(slot):
            @pl.loop(0, TOK_PER_WIN, step=16)
            def _(g):
                v = ibufs[slot][pl.ds(g, 16)]
                pbufs[slot][pl.ds(g, 16)] = lax.shift_right_logical(v, 1)
                hbufs[slot][pl.ds(g, 16)] = (v & 1) * HIDDEN

        def extract(c, slot, pslot):
            @pl.loop(0, CHUNK, step=16)
            def _(g):
                hv = hbufs[slot][pl.ds(c * CHUNK + g, 16)]
                for j in range(16):
                    t = c * CHUNK + g + j
                    h = hv[j]
                    for k in range(HIDDEN // 16):
                        obuf[t, pl.ds(16 * k, 16)] = (
                            pair[pslot, g + j, pl.ds(h + 16 * k, 16)])

        def window(win, slot):
            idx_copy(win, slot).wait()

            @pl.when(win + 1 < n_wins)
            def _():
                idx_copy(win + 1, 1 - slot).start()

            repack(slot)
            gather_start(0, slot, 0)

            @pl.when(win >= 1)
            def _():
                out_copy(win - 1).wait()

            for c in range(CHUNKS_PER_WIN):
                p = c % 2
                if c + 1 < CHUNKS_PER_WIN:
                    gather_start(c + 1, slot, 1 - p)
                gather_wait(p)
                extract(c, slot, p)

            out_copy(win).start()

        idx_copy(0, 0).start()

        @pl.loop(0, n_wins, step=2)
        def _(win):
            window(win, 0)
            window(win + 1, 1)

        out_copy(n_wins - 1).wait()

    out = gather_kernel(table2, idx)
    return out


# speedup vs baseline: 1.1910x; 1.0012x over previous
import functools

import jax
import jax.numpy as jnp
from jax import lax
from jax.experimental import pallas as pl
from jax.experimental.pallas import tpu as pltpu
from jax.experimental.pallas import tpu_sc as plsc

HIDDEN = 64
SEQ = 20
NUM_WORKERS = 32
ROWS_PER_WIN = 32
TOK_PER_WIN = ROWS_PER_WIN * SEQ
CHUNK = 128
CHUNKS_PER_WIN = TOK_PER_WIN // CHUNK


def kernel(input_ids, W):
    B, S = input_ids.shape
    n = B * S
    n_wins = B // (NUM_WORKERS * ROWS_PER_WIN)
    idx = input_ids.reshape(1, n).astype(jnp.int32)
    table2 = W.reshape(W.shape[0] // 2, 2 * HIDDEN)

    mesh = plsc.VectorSubcoreMesh(core_axis_name="core",
                                  subcore_axis_name="subcore")

    @functools.partial(
        pl.kernel,
        out_type=jax.ShapeDtypeStruct((B, S, HIDDEN), jnp.float32),
        mesh=mesh,
        scratch_types=[
            pltpu.VMEM((TOK_PER_WIN,), jnp.int32),
            pltpu.VMEM((TOK_PER_WIN,), jnp.int32),
            pltpu.VMEM((TOK_PER_WIN,), jnp.int32),
            pltpu.VMEM((TOK_PER_WIN,), jnp.int32),
            pltpu.VMEM((TOK_PER_WIN,), jnp.int32),
            pltpu.VMEM((TOK_PER_WIN,), jnp.int32),
            pltpu.VMEM((2, CHUNK, 2 * HIDDEN), jnp.float32),
            pltpu.VMEM((TOK_PER_WIN, HIDDEN), jnp.float32),
            pltpu.SemaphoreType.DMA,
            pltpu.SemaphoreType.DMA,
            pltpu.SemaphoreType.DMA,
        ],
    )
    def gather_kernel(table_hbm, idx_hbm, out_hbm,
                      ibuf0, ibuf1, pbuf0, pbuf1, hbuf0, hbuf1,
                      pair, obuf, isem, gsem, osem):
        ibufs, pbufs, hbufs = (ibuf0, ibuf1), (pbuf0, pbuf1), (hbuf0, hbuf1)
        wid = lax.axis_index("subcore") * 2 + lax.axis_index("core")
        base_row = wid * (n_wins * ROWS_PER_WIN)
        base_tok = base_row * SEQ

        def idx_copy(win, slot):
            return pltpu.make_async_copy(
                idx_hbm.at[0, pl.ds(base_tok + win * TOK_PER_WIN, TOK_PER_WIN)],
                ibufs[slot], isem)

        def gather_start(c, islot, pslot):
            for q in range(CHUNK // 16):
                pv = pbufs[islot][pl.ds(c * CHUNK + 16 * q, 16)]
                pltpu.make_async_copy(
                    table_hbm.at[pv],
                    pair.at[pslot, pl.ds(16 * q, 16)], gsem).start()

        def gather_wait(pslot):
            pltpu.make_async_copy(
                table_hbm.at[pbufs[0].at[pl.ds(0, CHUNK)]],
                pair.at[pslot], gsem).wait()

        def out_copy(win):
            return pltpu.make_async_copy(
                obuf.reshape(ROWS_PER_WIN, SEQ, HIDDEN),
                out_hbm.at[pl.ds(base_row + win * ROWS_PER_WIN, ROWS_PER_WIN)],
                osem)

        def repack(slot):
            @pl.loop(0, TOK_PER_WIN, step=16)
            def _(g):
                v = ibufs[slot][pl.ds(g, 16)]
                pbufs[slot][pl.ds(g, 16)] = lax.shift_right_logical(v, 1)
                hbufs[slot][pl.ds(g, 16)] = (v & 1) * HIDDEN

        def extract(c, slot, pslot):
            @pl.loop(0, CHUNK, step=16)
            def _(g):
                hv = hbufs[slot][pl.ds(c * CHUNK + g, 16)]
                for j in range(16):
                    t = c * CHUNK + g + j
                    h = hv[j]
                    for k in range(HIDDEN // 16):
                        obuf[t, pl.ds(16 * k, 16)] = (
                            pair[pslot, g + j, pl.ds(h + 16 * k, 16)])

        def window(win, slot):
            idx_copy(win, slot).wait()

            @pl.when(win + 1 < n_wins)
            def _():
                idx_copy(win + 1, 1 - slot).start()

            repack(slot)
            gather_start(0, slot, 0)

            @pl.when(win >= 1)
            def _():
                out_copy(win - 1).wait()

            for c in range(CHUNKS_PER_WIN):
                p = c % 2
                if c + 1 < CHUNKS_PER_WIN:
                    gather_start(c + 1, slot, 1 - p)
                gather_wait(p)
                extract(c, slot, p)

            out_copy(win).start()

        idx_copy(0, 0).start()

        @pl.loop(0, n_wins, step=2)
        def _(win):
            window(win, 0)
            window(win + 1, 1)

        out_copy(n_wins - 1).wait()

    out = gather_kernel(table2, idx)
    return out


# speedup vs baseline: 1.3276x; 1.1147x over previous
import functools

import jax
import jax.numpy as jnp
from jax import lax
from jax.experimental import pallas as pl
from jax.experimental.pallas import tpu as pltpu
from jax.experimental.pallas import tpu_sc as plsc

HIDDEN = 64
SEQ = 20
NUM_WORKERS = 32
ROWS_PER_WIN = 32
TOK_PER_WIN = ROWS_PER_WIN * SEQ
CHUNK = 128
CHUNKS_PER_WIN = TOK_PER_WIN // CHUNK


def kernel(input_ids, W):
    B, S = input_ids.shape
    n = B * S
    n_wins = B // (NUM_WORKERS * ROWS_PER_WIN)
    idx = input_ids.astype(jnp.int32)
    table2 = W.reshape(W.shape[0] // 2, 2 * HIDDEN)

    mesh = plsc.VectorSubcoreMesh(core_axis_name="core",
                                  subcore_axis_name="subcore")

    @functools.partial(
        pl.kernel,
        out_type=jax.ShapeDtypeStruct((B, S * HIDDEN), jnp.float32),
        mesh=mesh,
        scratch_types=[
            pltpu.VMEM((ROWS_PER_WIN, SEQ), jnp.int32),
            pltpu.VMEM((ROWS_PER_WIN, SEQ), jnp.int32),
            pltpu.VMEM((TOK_PER_WIN,), jnp.int32),
            pltpu.VMEM((TOK_PER_WIN,), jnp.int32),
            pltpu.VMEM((TOK_PER_WIN,), jnp.int32),
            pltpu.VMEM((TOK_PER_WIN,), jnp.int32),
            pltpu.VMEM((2, CHUNK, 2 * HIDDEN), jnp.float32),
            pltpu.VMEM((ROWS_PER_WIN, SEQ * HIDDEN), jnp.float32),
            pltpu.SemaphoreType.DMA,
            pltpu.SemaphoreType.DMA,
            pltpu.SemaphoreType.DMA,
        ],
    )
    def gather_kernel(table_hbm, idx_hbm, out_hbm,
                      ibuf0, ibuf1, pbuf0, pbuf1, hbuf0, hbuf1,
                      pair, obuf, isem, gsem, osem):
        ibufs, pbufs, hbufs = (ibuf0, ibuf1), (pbuf0, pbuf1), (hbuf0, hbuf1)
        wid = lax.axis_index("subcore") * 2 + lax.axis_index("core")
        base_row = wid * (n_wins * ROWS_PER_WIN)
        base_tok = base_row * SEQ

        def idx_copy(win, slot):
            return pltpu.make_async_copy(
                idx_hbm.at[pl.ds(base_row + win * ROWS_PER_WIN, ROWS_PER_WIN)],
                ibufs[slot], isem)

        def gather_start(c, islot, pslot):
            for q in range(CHUNK // 16):
                pv = pbufs[islot][pl.ds(c * CHUNK + 16 * q, 16)]
                pltpu.make_async_copy(
                    table_hbm.at[pv],
                    pair.at[pslot, pl.ds(16 * q, 16)], gsem).start()

        def gather_wait(pslot):
            pltpu.make_async_copy(
                table_hbm.at[pbufs[0].at[pl.ds(0, CHUNK)]],
                pair.at[pslot], gsem).wait()

        def out_copy(win):
            return pltpu.make_async_copy(
                obuf,
                out_hbm.at[pl.ds(base_row + win * ROWS_PER_WIN, ROWS_PER_WIN)],
                osem)

        def repack(slot):
            for r in range(ROWS_PER_WIN):
                for o in (0, SEQ - 16):
                    v = ibufs[slot][r, pl.ds(o, 16)]
                    pbufs[slot][pl.ds(r * SEQ + o, 16)] = (
                        lax.shift_right_logical(v, 1))
                    hbufs[slot][pl.ds(r * SEQ + o, 16)] = (v & 1) * HIDDEN

        def extract(c, slot, pslot):
            @pl.loop(0, CHUNK, step=16)
            def _(g):
                hv = hbufs[slot][pl.ds(c * CHUNK + g, 16)]
                for j in range(16):
                    t = c * CHUNK + g + j
                    b = t // SEQ
                    col0 = (t - b * SEQ) * HIDDEN
                    h = hv[j]
                    for k in range(HIDDEN // 16):
                        obuf[b, pl.ds(col0 + 16 * k, 16)] = (
                            pair[pslot, g + j, pl.ds(h + 16 * k, 16)])

        def window(win, slot):
            idx_copy(win, slot).wait()

            @pl.when(win + 1 < n_wins)
            def _():
                idx_copy(win + 1, 1 - slot).start()

            repack(slot)
            gather_start(0, slot, 0)

            @pl.when(win >= 1)
            def _():
                out_copy(win - 1).wait()

            for c in range(CHUNKS_PER_WIN):
                p = c % 2
                if c + 1 < CHUNKS_PER_WIN:
                    gather_start(c + 1, slot, 1 - p)
                gather_wait(p)
                extract(c, slot, p)

            out_copy(win).start()

        idx_copy(0, 0).start()

        @pl.loop(0, n_wins, step=2)
        def _(win):
            window(win, 0)
            window(win + 1, 1)

        out_copy(n_wins - 1).wait()

    out = gather_kernel(table2, idx)
    return out.reshape(B, S, HIDDEN)
